# baseline (speedup 1.0000x reference)
import functools

import jax
import jax.numpy as jnp
from jax import lax
from jax.experimental import pallas as pl
from jax.experimental.pallas import tpu as pltpu
from jax.experimental.pallas import tpu_sc as plsc

BATCH = 16384
EMBED_DIM = 32
NUM_CORES = 2
NUM_SUBCORES = 16
NUM_WORKERS = NUM_CORES * NUM_SUBCORES
ROWS_PER_WORKER = BATCH // NUM_WORKERS
CHUNK = 128
NUM_CHUNKS = ROWS_PER_WORKER // CHUNK

_sc_mesh = plsc.VectorSubcoreMesh(core_axis_name="c", subcore_axis_name="s")


@functools.partial(
    pl.kernel,
    mesh=_sc_mesh,
    out_type=[
        jax.ShapeDtypeStruct((EMBED_DIM, BATCH), jnp.float32),
        jax.ShapeDtypeStruct((EMBED_DIM, BATCH), jnp.float32),
    ],
    scratch_types=[
        pltpu.VMEM((ROWS_PER_WORKER,), jnp.int32),
        pltpu.VMEM((ROWS_PER_WORKER,), jnp.int32),
        pltpu.VMEM((EMBED_DIM, ROWS_PER_WORKER), jnp.float32),
        pltpu.VMEM((EMBED_DIM, ROWS_PER_WORKER), jnp.float32),
        pltpu.SemaphoreType.DMA,
    ],
    compiler_params=pltpu.CompilerParams(use_tc_tiling_on_sc=False),
)
def _sc_gather(uids_hbm, iids_hbm, utabT_hbm, itabT_hbm, uoutT_hbm, ioutT_hbm,
               uidx_v, iidx_v, uT_v, iT_v, sem):
    wid = lax.axis_index("s") * NUM_CORES + lax.axis_index("c")
    base = wid * ROWS_PER_WORKER
    pltpu.sync_copy(uids_hbm.at[wid], uidx_v)
    pltpu.sync_copy(iids_hbm.at[wid], iidx_v)
    copies = []
    for c in range(EMBED_DIM):
        for j in range(NUM_CHUNKS):
            sl = pl.ds(j * CHUNK, CHUNK)
            copies.append(
                pltpu.async_copy(
                    utabT_hbm.at[c].at[uidx_v.at[sl]], uT_v.at[c].at[sl], sem))
            copies.append(
                pltpu.async_copy(
                    itabT_hbm.at[c].at[iidx_v.at[sl]], iT_v.at[c].at[sl], sem))
    for cp in copies:
        cp.wait()
    dst = pl.ds(base, ROWS_PER_WORKER)
    pltpu.sync_copy(uT_v, uoutT_hbm.at[:, dst])
    pltpu.sync_copy(iT_v, ioutT_hbm.at[:, dst])


MLP_BLOCK = 2048


def _mlp_body(u_ref, i_ref, w1u_ref, w1i_ref, b1_ref, w2_ref, b2_ref,
              w3_ref, b3_ref, o_ref):
    h = jnp.dot(w1u_ref[...], u_ref[...], preferred_element_type=jnp.float32)
    h = h + jnp.dot(w1i_ref[...], i_ref[...], preferred_element_type=jnp.float32)
    h = jnp.maximum(h + b1_ref[...], 0.0)
    h = jnp.dot(w2_ref[...], h, preferred_element_type=jnp.float32) + b2_ref[...]
    h = jnp.maximum(h, 0.0)
    o_ref[...] = (
        jnp.dot(w3_ref[...], h, preferred_element_type=jnp.float32) + b3_ref[...]
    )


def _mlp(uT, iT, W1, b1, W2, b2, W3, b3):
    w1uT = W1[:EMBED_DIM].T
    w1iT = W1[EMBED_DIM:].T
    grid = (BATCH // MLP_BLOCK,)
    full = lambda shape: pl.BlockSpec(shape, lambda i: (0, 0))
    out = pl.pallas_call(
        _mlp_body,
        grid=grid,
        in_specs=[
            pl.BlockSpec((EMBED_DIM, MLP_BLOCK), lambda i: (0, i)),
            pl.BlockSpec((EMBED_DIM, MLP_BLOCK), lambda i: (0, i)),
            full((64, EMBED_DIM)),
            full((64, EMBED_DIM)),
            full((64, 1)),
            full((32, 64)),
            full((32, 1)),
            full((1, 32)),
            full((1, 1)),
        ],
        out_specs=pl.BlockSpec((1, MLP_BLOCK), lambda i: (0, i)),
        out_shape=jax.ShapeDtypeStruct((1, BATCH), jnp.float32),
    )(uT, iT, w1uT, w1iT, b1.reshape(64, 1), W2.T, b2.reshape(32, 1),
      W3.T, b3.reshape(1, 1))
    return out[0]


def kernel(user_ids, item_ids, user_table, item_table, W1, b1, W2, b2, W3, b3):
    uids = user_ids.astype(jnp.int32)
    iids = item_ids.astype(jnp.int32)
    uT, iT = _sc_gather(
        uids.reshape(NUM_WORKERS, ROWS_PER_WORKER),
        iids.reshape(NUM_WORKERS, ROWS_PER_WORKER),
        user_table.T, item_table.T)
    return _mlp(uT, iT, W1, b1, W2, b2, W3, b3)
